# baseline (speedup 1.0000x reference)
import functools

import jax
import jax.numpy as jnp
from jax import lax
from jax.experimental import pallas as pl
from jax.experimental.pallas import tpu as pltpu
from jax.experimental.pallas import tpu_sc as plsc

_CYCLE = 168
_LEN = 336
_BATCH = 1024
_D = 128

_NC = 2
_NS = 16
_NW = _NC * _NS
_BPW = _BATCH // _NW


_K_STREAM = 20


@functools.partial(
    pl.kernel,
    mesh=plsc.VectorSubcoreMesh(core_axis_name="c", subcore_axis_name="s"),
    out_type=jax.ShapeDtypeStruct((_BATCH, _LEN, _D), jnp.float32),
    scratch_types=[
        pltpu.VMEM((3 * _CYCLE, _D), jnp.float32),
        pltpu.VMEM((_BPW,), jnp.int32),
        pltpu.VMEM_SHARED((3 * _CYCLE, _D), jnp.float32),
        pltpu.SemaphoreType.DMA,
        pltpu.SemaphoreType.DMA,
    ],
)
def _cycle_gather(data_hbm, idx_hbm, out_hbm, table_v, idx_v, table_s, sem, sem2):
    sid = lax.axis_index("s")
    wid = lax.axis_index("c") * _NS + sid
    base = wid * _BPW
    stage = [
        pltpu.make_async_copy(data_hbm, table_v.at[pl.ds(k * _CYCLE, _CYCLE)], sem)
        for k in range(3)
    ]
    stage.append(pltpu.make_async_copy(idx_hbm.at[pl.ds(base, _BPW)], idx_v, sem))
    for c in stage:
        c.start()

    @pl.when(sid == 0)
    def _():
        s_stage = [
            pltpu.make_async_copy(data_hbm, table_s.at[pl.ds(k * _CYCLE, _CYCLE)], sem2)
            for k in range(3)
        ]
        for c in s_stage:
            c.start()
        for c in s_stage:
            c.wait()

    for c in stage:
        c.wait()

    vecs = [idx_v[pl.ds(h * 16, 16)] for h in range(_BPW // 16)]
    starts = [vecs[j // 16][j % 16] for j in range(_BPW)]

    copies = []
    for j in range(_K_STREAM):
        c = pltpu.make_async_copy(
            table_v.at[pl.ds(starts[j], _LEN)],
            out_hbm.at[base + j],
            sem,
        )
        c.start()
        copies.append(c)

    plsc.subcore_barrier()

    for j in range(_K_STREAM, _BPW):
        c = pltpu.make_async_copy(
            table_s.at[pl.ds(starts[j], _LEN)],
            out_hbm.at[base + j],
            sem2,
        )
        c.start()
        copies.append(c)
    for c in copies:
        c.wait()


def kernel(index, length, data):
    shift = jnp.mod(jnp.asarray(length, jnp.int32) - _LEN, _CYCLE)
    idx = jnp.mod(index.astype(jnp.int32) + shift, _CYCLE)
    return _cycle_gather(data, idx)
